# baseline (speedup 1.0000x reference)
import jax
import jax.numpy as jnp
from jax.experimental import pallas as pl

_IMAGE_SIZE = 224
_PATCH = 16
_C = 3
_HIDDEN = 768
_GRID = _IMAGE_SIZE // _PATCH
_NUM_PATCHES = _GRID * _GRID
_LEN_KEEP = 49
_KPAD = 64
_PVEC = _PATCH * _PATCH * _C


def _embed_kernel(nrow_ref, ncol_ref, x_ref, w_ref, pos_ref, cls_ref, pos0_ref,
                  b_ref, out_ref, mask_ref, ids_ref):
    nrow = nrow_ref[0]
    ncol = ncol_ref[0]
    ii = jax.lax.broadcasted_iota(jnp.int32, (_NUM_PATCHES, _NUM_PATCHES), 1)
    jj = jax.lax.broadcasted_iota(jnp.int32, (_NUM_PATCHES, _NUM_PATCHES), 0)
    less = (nrow < ncol) | ((nrow == ncol) & (ii < jj))
    rank = jnp.sum(less.astype(jnp.int32), axis=1, keepdims=True)
    rank_row = jnp.sum(less.astype(jnp.int32), axis=1)
    ids_ref[0, 0, :] = rank_row
    mask_ref[0, 0, :] = (rank_row >= _LEN_KEEP).astype(jnp.float32)

    kio = jax.lax.broadcasted_iota(jnp.int32, (_KPAD, _NUM_PATCHES), 0)
    P = (kio == rank_row[None, :]).astype(jnp.float32)

    xk = jnp.dot(P, x_ref[0], preferred_element_type=jnp.float32)
    posk = jnp.dot(P, pos_ref[...], preferred_element_type=jnp.float32)
    emb = jnp.dot(xk, w_ref[...], preferred_element_type=jnp.float32)
    emb = emb + b_ref[...] + posk
    out_ref[0, 0, :] = cls_ref[0, :] + pos0_ref[0, :]
    out_ref[0, 1:, :] = emb[:_LEN_KEEP, :]


def kernel(pixel_values, noise, proj_W, proj_b, cls_token, pos_embed):
    B = pixel_values.shape[0]
    x = jnp.transpose(pixel_values, (0, 2, 3, 1))
    x = x.reshape(B, _GRID, _PATCH, _GRID, _PATCH, _C)
    x = jnp.transpose(x, (0, 1, 3, 2, 4, 5)).reshape(B, _NUM_PATCHES, _PVEC)
    Wm = proj_W.reshape(_PVEC, _HIDDEN)
    pos_seq = pos_embed[0, 1:, :]
    pos0 = pos_embed[:, 0, :]
    cls = cls_token.reshape(1, _HIDDEN)
    bias = proj_b.reshape(1, _HIDDEN)
    noise_row = noise.reshape(B, 1, _NUM_PATCHES)
    noise_col = noise.reshape(B, _NUM_PATCHES, 1)

    out, mask, ids = pl.pallas_call(
        _embed_kernel,
        grid=(B,),
        in_specs=[
            pl.BlockSpec((1, 1, _NUM_PATCHES), lambda b: (b, 0, 0)),
            pl.BlockSpec((1, _NUM_PATCHES, 1), lambda b: (b, 0, 0)),
            pl.BlockSpec((1, _NUM_PATCHES, _PVEC), lambda b: (b, 0, 0)),
            pl.BlockSpec((_PVEC, _HIDDEN), lambda b: (0, 0)),
            pl.BlockSpec((_NUM_PATCHES, _HIDDEN), lambda b: (0, 0)),
            pl.BlockSpec((1, _HIDDEN), lambda b: (0, 0)),
            pl.BlockSpec((1, _HIDDEN), lambda b: (0, 0)),
            pl.BlockSpec((1, _HIDDEN), lambda b: (0, 0)),
        ],
        out_specs=[
            pl.BlockSpec((1, 1 + _LEN_KEEP, _HIDDEN), lambda b: (b, 0, 0)),
            pl.BlockSpec((1, 1, _NUM_PATCHES), lambda b: (b, 0, 0)),
            pl.BlockSpec((1, 1, _NUM_PATCHES), lambda b: (b, 0, 0)),
        ],
        out_shape=[
            jax.ShapeDtypeStruct((B, 1 + _LEN_KEEP, _HIDDEN), jnp.float32),
            jax.ShapeDtypeStruct((B, 1, _NUM_PATCHES), jnp.float32),
            jax.ShapeDtypeStruct((B, 1, _NUM_PATCHES), jnp.int32),
        ],
    )(noise_row, noise_col, x, Wm, pos_seq, cls, pos0, bias)
    return (out, mask.reshape(B, _NUM_PATCHES), ids.reshape(B, _NUM_PATCHES))


# speedup vs baseline: 1.2900x; 1.2900x over previous
import jax
import jax.numpy as jnp
from jax.experimental import pallas as pl

_IMAGE_SIZE = 224
_PATCH = 16
_C = 3
_HIDDEN = 768
_GRID = _IMAGE_SIZE // _PATCH
_NUM_PATCHES = _GRID * _GRID
_LEN_KEEP = 49
_KPAD = 64
_PVEC = _PATCH * _PATCH * _C
_G = 4


def _embed_kernel(nrow_ref, ncol_ref, x_ref, w_ref, pos_ref, b_ref, cls_ref,
                  pos0_ref, out_ref, mask_ref, ids_ref):
    nrow = nrow_ref[...]
    ncol = ncol_ref[...]
    ii = jax.lax.broadcasted_iota(jnp.int32, (_G, _NUM_PATCHES, _NUM_PATCHES), 2)
    jj = jax.lax.broadcasted_iota(jnp.int32, (_G, _NUM_PATCHES, _NUM_PATCHES), 1)
    less = (nrow < ncol) | ((nrow == ncol) & (ii < jj))
    rank = jnp.sum(less.astype(jnp.int32), axis=2)
    ids_ref[:, 0, :] = rank
    mask_ref[:, 0, :] = (rank >= _LEN_KEEP).astype(jnp.float32)

    kio = jax.lax.broadcasted_iota(jnp.int32, (_G, _KPAD, _NUM_PATCHES), 1)
    P = (kio == rank[:, None, :]).astype(jnp.bfloat16)

    xs = x_ref[...]
    xk = jnp.concatenate(
        [jnp.dot(P[s], xs[s], preferred_element_type=jnp.float32).astype(jnp.bfloat16)
         for s in range(_G)], axis=0)
    Ps = P.reshape(_G * _KPAD, _NUM_PATCHES)

    pos_aug = pos_ref[...] + b_ref[...]
    emb = (jnp.dot(xk, w_ref[...], preferred_element_type=jnp.float32)
           + jnp.dot(Ps, pos_aug, preferred_element_type=jnp.float32))
    cls_row = cls_ref[...] + pos0_ref[...]
    for s in range(_G):
        out_ref[s, 0, :] = cls_row[0]
        out_ref[s, 1:, :] = emb[s * _KPAD:s * _KPAD + _LEN_KEEP, :]


def kernel(pixel_values, noise, proj_W, proj_b, cls_token, pos_embed):
    B = pixel_values.shape[0]
    x = pixel_values.reshape(B, _C, _GRID, _PATCH, _GRID, _PATCH)
    x = x.transpose(0, 2, 4, 3, 5, 1).reshape(B, _NUM_PATCHES, _PVEC)
    x = x.astype(jnp.bfloat16)
    Wm = proj_W.reshape(_PVEC, _HIDDEN).astype(jnp.bfloat16)
    pos_seq = pos_embed[0, 1:, :].astype(jnp.bfloat16)
    pos0 = pos_embed[:, 0, :]
    cls = cls_token.reshape(1, _HIDDEN)
    bias = proj_b.reshape(1, _HIDDEN).astype(jnp.bfloat16)
    noise_row = noise.reshape(B, 1, _NUM_PATCHES)
    noise_col = noise.reshape(B, _NUM_PATCHES, 1)

    out, mask, ids = pl.pallas_call(
        _embed_kernel,
        grid=(B // _G,),
        in_specs=[
            pl.BlockSpec((_G, 1, _NUM_PATCHES), lambda g: (g, 0, 0)),
            pl.BlockSpec((_G, _NUM_PATCHES, 1), lambda g: (g, 0, 0)),
            pl.BlockSpec((_G, _NUM_PATCHES, _PVEC), lambda g: (g, 0, 0)),
            pl.BlockSpec((_PVEC, _HIDDEN), lambda g: (0, 0)),
            pl.BlockSpec((_NUM_PATCHES, _HIDDEN), lambda g: (0, 0)),
            pl.BlockSpec((1, _HIDDEN), lambda g: (0, 0)),
            pl.BlockSpec((1, _HIDDEN), lambda g: (0, 0)),
            pl.BlockSpec((1, _HIDDEN), lambda g: (0, 0)),
        ],
        out_specs=[
            pl.BlockSpec((_G, 1 + _LEN_KEEP, _HIDDEN), lambda g: (g, 0, 0)),
            pl.BlockSpec((_G, 1, _NUM_PATCHES), lambda g: (g, 0, 0)),
            pl.BlockSpec((_G, 1, _NUM_PATCHES), lambda g: (g, 0, 0)),
        ],
        out_shape=[
            jax.ShapeDtypeStruct((B, 1 + _LEN_KEEP, _HIDDEN), jnp.float32),
            jax.ShapeDtypeStruct((B, 1, _NUM_PATCHES), jnp.float32),
            jax.ShapeDtypeStruct((B, 1, _NUM_PATCHES), jnp.int32),
        ],
    )(noise_row, noise_col, x, Wm, pos_seq, bias, cls, pos0)
    return (out, mask.reshape(B, _NUM_PATCHES), ids.reshape(B, _NUM_PATCHES))
